# baseline (speedup 1.0000x reference)
import numpy as np
import jax
import jax.numpy as jnp
from jax.experimental import pallas as pl


def _hadamard(n: int = 256) -> np.ndarray:
    i = np.arange(n)
    a = np.bitwise_and(i[:, None], i[None, :])
    p = a ^ (a >> 4)
    p ^= p >> 2
    p ^= p >> 1
    return (1.0 - 2.0 * (p & 1)).astype(np.float32)


_H256 = _hadamard(256)


def _body(t_ref, p_ref, m_ref, h_ref, o_ref):
    H = h_ref[...]

    def dot(a, b):
        return jax.lax.dot_general(
            a, b, (((1,), (0,)), ((), ())),
            preferred_element_type=jnp.float32,
        )

    ht = dot(t_ref[...].astype(jnp.bfloat16), H)
    hp = dot(p_ref[...].astype(jnp.bfloat16), H)
    prod = ht * hp
    s = prod.reshape(-1, 16, 256).sum(axis=1)
    y_true = dot(s.astype(jnp.bfloat16), H) * (1.0 / 4096.0)

    om = m_ref[...]
    omn = om / jnp.sum(om, axis=-1, keepdims=True)
    omn = jnp.clip(omn, 1e-7, 1.0 - 1e-7)
    o_ref[0, 0, :] = -jnp.sum(y_true * jnp.log(omn), axis=-1)


def kernel(targets, predictions, output_mask, xor_map1, xor_map2):
    del xor_map1, xor_map2
    B, NB, C = targets.shape
    BLK = 64
    grid = B // BLK
    tf = targets.reshape(B * NB, C)
    pf = predictions.reshape(B * NB, C)
    h = jnp.asarray(_H256, dtype=jnp.bfloat16)
    out = pl.pallas_call(
        _body,
        grid=(grid,),
        in_specs=[
            pl.BlockSpec((BLK * NB, C), lambda i: (i, 0)),
            pl.BlockSpec((BLK * NB, C), lambda i: (i, 0)),
            pl.BlockSpec((BLK, C), lambda i: (i, 0)),
            pl.BlockSpec((C, C), lambda i: (0, 0)),
        ],
        out_specs=pl.BlockSpec((1, 1, BLK), lambda i: (i, 0, 0)),
        out_shape=jax.ShapeDtypeStruct((grid, 1, BLK), jnp.float32),
    )(tf, pf, output_mask, h)
    return out.reshape(B)


# speedup vs baseline: 1.5610x; 1.5610x over previous
import numpy as np
import jax
import jax.numpy as jnp
from jax.experimental import pallas as pl


def _hadamard(n: int = 256) -> np.ndarray:
    i = np.arange(n)
    a = np.bitwise_and(i[:, None], i[None, :])
    p = a ^ (a >> 4)
    p ^= p >> 2
    p ^= p >> 1
    return (1.0 - 2.0 * (p & 1)).astype(np.float32)


_H256 = _hadamard(256)


def _body(t_ref, p_ref, m_ref, h_ref, o_ref):
    H = h_ref[...]

    def dot(a, b):
        return jax.lax.dot_general(
            a, b, (((1,), (0,)), ((), ())),
            preferred_element_type=jnp.float32,
        )

    ht = dot(t_ref[...].astype(jnp.bfloat16), H)
    hp = dot(p_ref[...].astype(jnp.bfloat16), H)
    prod = ht * hp
    s = prod.reshape(-1, 16, 256).sum(axis=1)
    y_true = dot(s.astype(jnp.bfloat16), H) * (1.0 / 4096.0)

    om = m_ref[...]
    omn = om / jnp.sum(om, axis=-1, keepdims=True)
    omn = jnp.clip(omn, 1e-7, 1.0 - 1e-7)
    o_ref[0, 0, :] = -jnp.sum(y_true * jnp.log(omn), axis=-1)


def kernel(targets, predictions, output_mask, xor_map1, xor_map2):
    del xor_map1, xor_map2
    B, NB, C = targets.shape
    BLK = 256
    grid = B // BLK
    tf = targets.reshape(B * NB, C)
    pf = predictions.reshape(B * NB, C)
    h = jnp.asarray(_H256, dtype=jnp.bfloat16)
    out = pl.pallas_call(
        _body,
        grid=(grid,),
        in_specs=[
            pl.BlockSpec((BLK * NB, C), lambda i: (i, 0)),
            pl.BlockSpec((BLK * NB, C), lambda i: (i, 0)),
            pl.BlockSpec((BLK, C), lambda i: (i, 0)),
            pl.BlockSpec((C, C), lambda i: (0, 0)),
        ],
        out_specs=pl.BlockSpec((1, 1, BLK), lambda i: (i, 0, 0)),
        out_shape=jax.ShapeDtypeStruct((grid, 1, BLK), jnp.float32),
    )(tf, pf, output_mask, h)
    return out.reshape(B)
